# baseline (speedup 1.0000x reference)
import functools

import jax
import jax.numpy as jnp
from jax import lax
from jax.experimental import pallas as pl
from jax.experimental.pallas import tpu as pltpu
from jax.experimental.pallas import tpu_sc as plsc

_NC = 2
_NS = 16
_NW = _NC * _NS
_L = 16
_D = 32


def _sc_body(uids_hbm, iids_hbm, utab_hbm, itab_hbm, out_hbm,
             uidx_v, iidx_v, urows_v, irows_v, out_v, sem_u, sem_i):
    n = uidx_v.shape[0]
    wid = lax.axis_index("s") * _NC + lax.axis_index("c")
    base = wid * n
    pltpu.sync_copy(uids_hbm.at[pl.ds(base, n)], uidx_v)
    pltpu.sync_copy(iids_hbm.at[pl.ds(base, n)], iidx_v)
    cu = pltpu.async_copy(utab_hbm.at[uidx_v], urows_v, sem_u)
    ci = pltpu.async_copy(itab_hbm.at[iidx_v], irows_v, sem_i)
    cu.wait()
    ci.wait()

    iota = lax.iota(jnp.int32, _L)

    def g_body(g, carry):
        rows = g * _L + iota
        acc = jnp.zeros((_L,), jnp.float32)
        for d in range(_D):
            dvec = jnp.full((_L,), d, jnp.int32)
            uu = plsc.load_gather(urows_v, [rows, dvec])
            vv = plsc.load_gather(irows_v, [rows, dvec])
            acc = acc + uu * vv
        out_v[pl.ds(g * _L, _L)] = acc
        return carry

    lax.fori_loop(0, n // _L, g_body, 0)
    pltpu.sync_copy(out_v, out_hbm.at[pl.ds(base, n)])


@jax.jit
def kernel(user_ids, item_ids, user_table, item_table):
    B = user_ids.shape[0]
    n = B // _NW
    mesh = plsc.VectorSubcoreMesh(core_axis_name="c", subcore_axis_name="s")
    k = pl.kernel(
        _sc_body,
        out_type=jax.ShapeDtypeStruct((B,), jnp.float32),
        mesh=mesh,
        scratch_types=[
            pltpu.VMEM((n,), jnp.int32),
            pltpu.VMEM((n,), jnp.int32),
            pltpu.VMEM((n, _D), jnp.float32),
            pltpu.VMEM((n, _D), jnp.float32),
            pltpu.VMEM((n,), jnp.float32),
            pltpu.SemaphoreType.DMA,
            pltpu.SemaphoreType.DMA,
        ],
        compiler_params=pltpu.CompilerParams(
            needs_layout_passes=False, use_tc_tiling_on_sc=False
        ),
    )
    return k(user_ids, item_ids, user_table, item_table)
